# baseline (speedup 1.0000x reference)
import jax
import jax.numpy as jnp
from jax import lax
from jax.experimental import pallas as pl
from jax.experimental.pallas import tpu as pltpu
from jax.experimental.pallas import tpu_sc as plsc

TIME_DIM = 64
HOUR_SIZE = 24
DAY_SIZE = 32
MONTH_SIZE = 13
BATCH = 16384

NC = 2
NS = 16
L = 16
NW = NC * NS
B_PER_W = BATCH // NW
N_CHUNKS = B_PER_W // L

HOUR_OFF = 0
DAY_OFF = HOUR_SIZE * TIME_DIM
MONTH_OFF = DAY_OFF + DAY_SIZE * TIME_DIM
TABLE_WORDS = MONTH_OFF + MONTH_SIZE * TIME_DIM


def _body(x_hbm, tab_hbm, out_hbm, xv, tv, ov, ah, ad, am, sem):
    wid = lax.axis_index("s") * NC + lax.axis_index("c")
    base = wid * B_PER_W

    cp_x = pltpu.async_copy(x_hbm.at[pl.ds(base * 3, B_PER_W * 3)], xv, sem)
    cp_t = pltpu.async_copy(tab_hbm, tv, sem)
    for cp in (cp_x, cp_t):
        cp.wait()

    lane = lax.iota(jnp.int32, L)
    lane3 = lane * 3

    @plsc.parallel_loop(0, N_CHUNKS, unroll=2)
    def _chunk(c):
        b0 = c * (L * 3)
        vm = plsc.load_gather(xv, [lane3 + b0])
        vd = plsc.load_gather(xv, [lane3 + (b0 + 1)])
        vh = plsc.load_gather(xv, [lane3 + (b0 + 2)])
        sl = pl.ds(c * L, L)
        ah[sl] = (vh * HOUR_SIZE).astype(jnp.int32) * TIME_DIM + HOUR_OFF
        ad[sl] = (vd * DAY_SIZE).astype(jnp.int32) * TIME_DIM + DAY_OFF
        am[sl] = (vm * MONTH_SIZE).astype(jnp.int32) * TIME_DIM + MONTH_OFF

    @plsc.parallel_loop(0, B_PER_W, unroll=2)
    def _row(r):
        rv = jnp.full((L,), 0, jnp.int32) + r
        hb = plsc.load_gather(ah, [rv]) + lane
        db = plsc.load_gather(ad, [rv]) + lane
        mb = plsc.load_gather(am, [rv]) + lane
        for g in range(TIME_DIM // L):
            va = plsc.load_gather(tv, [hb + g * L])
            vb = plsc.load_gather(tv, [db + g * L])
            vc = plsc.load_gather(tv, [mb + g * L])
            ov[r, pl.ds(g * L, L)] = (va + vb) + vc

    pltpu.sync_copy(ov, out_hbm.at[pl.ds(base, B_PER_W)])


@jax.jit
def kernel(x, hour_table, day_table, month_table):
    run = pl.kernel(
        _body,
        out_type=jax.ShapeDtypeStruct((BATCH, TIME_DIM), jnp.float32),
        mesh=plsc.VectorSubcoreMesh(
            core_axis_name="c", subcore_axis_name="s",
            num_cores=NC, num_subcores=NS),
        scratch_types=[
            pltpu.VMEM((B_PER_W * 3,), jnp.float32),
            pltpu.VMEM((TABLE_WORDS,), jnp.float32),
            pltpu.VMEM((B_PER_W, TIME_DIM), jnp.float32),
            pltpu.VMEM((B_PER_W,), jnp.int32),
            pltpu.VMEM((B_PER_W,), jnp.int32),
            pltpu.VMEM((B_PER_W,), jnp.int32),
            pltpu.SemaphoreType.DMA,
        ],
        compiler_params=pltpu.CompilerParams(
            needs_layout_passes=False, use_tc_tiling_on_sc=False,
            disable_bounds_checks=True, disable_semaphore_checks=True,
            skip_device_barrier=True),
    )
    tab = jnp.concatenate(
        [hour_table.reshape(-1), day_table.reshape(-1),
         month_table.reshape(-1)])
    return run(x.reshape(-1), tab)


# speedup vs baseline: 1.0042x; 1.0042x over previous
import functools

import jax
import jax.numpy as jnp
from jax import lax
from jax.experimental import layout as jex_layout
from jax.experimental import pallas as pl
from jax.experimental.pallas import tpu as pltpu
from jax.experimental.pallas import tpu_sc as plsc

TIME_DIM = 64
HOUR_SIZE = 24
DAY_SIZE = 32
MONTH_SIZE = 13
BATCH = 16384

NC = 2
NS = 16
L = 16
NW = NC * NS
B_PER_W = BATCH // NW
N_CHUNKS = B_PER_W // L

HOUR_OFF = 0
DAY_OFF = HOUR_SIZE * TIME_DIM
MONTH_OFF = DAY_OFF + DAY_SIZE * TIME_DIM
TABLE_WORDS = MONTH_OFF + MONTH_SIZE * TIME_DIM


def _body(x_hbm, tab_hbm, out_hbm, xv, tv, ov, ah, ad, am, sem):
    wid = lax.axis_index("s") * NC + lax.axis_index("c")
    base = wid * B_PER_W

    cp_x = pltpu.async_copy(x_hbm.at[pl.ds(base * 3, B_PER_W * 3)], xv, sem)
    cp_t = pltpu.async_copy(tab_hbm, tv, sem)
    for cp in (cp_x, cp_t):
        cp.wait()

    lane = lax.iota(jnp.int32, L)
    lane3 = lane * 3

    @plsc.parallel_loop(0, N_CHUNKS, unroll=2)
    def _chunk(c):
        b0 = c * (L * 3)
        vm = plsc.load_gather(xv, [lane3 + b0])
        vd = plsc.load_gather(xv, [lane3 + (b0 + 1)])
        vh = plsc.load_gather(xv, [lane3 + (b0 + 2)])
        sl = pl.ds(c * L, L)
        ah[sl] = (vh * HOUR_SIZE).astype(jnp.int32) * TIME_DIM + HOUR_OFF
        ad[sl] = (vd * DAY_SIZE).astype(jnp.int32) * TIME_DIM + DAY_OFF
        am[sl] = (vm * MONTH_SIZE).astype(jnp.int32) * TIME_DIM + MONTH_OFF

    @plsc.parallel_loop(0, B_PER_W, unroll=2)
    def _row(r):
        rv = jnp.full((L,), 0, jnp.int32) + r
        hb = plsc.load_gather(ah, [rv]) + lane
        db = plsc.load_gather(ad, [rv]) + lane
        mb = plsc.load_gather(am, [rv]) + lane
        for g in range(TIME_DIM // L):
            va = plsc.load_gather(tv, [hb + g * L])
            vb = plsc.load_gather(tv, [db + g * L])
            vc = plsc.load_gather(tv, [mb + g * L])
            ov[r, pl.ds(g * L, L)] = (va + vb) + vc

    pltpu.sync_copy(ov, out_hbm.at[pl.ds(base, B_PER_W)])


def _kernel_impl(x, hour_table, day_table, month_table):
    run = pl.kernel(
        _body,
        out_type=jax.ShapeDtypeStruct((BATCH, TIME_DIM), jnp.float32),
        mesh=plsc.VectorSubcoreMesh(
            core_axis_name="c", subcore_axis_name="s",
            num_cores=NC, num_subcores=NS),
        scratch_types=[
            pltpu.VMEM((B_PER_W * 3,), jnp.float32),
            pltpu.VMEM((TABLE_WORDS,), jnp.float32),
            pltpu.VMEM((B_PER_W, TIME_DIM), jnp.float32),
            pltpu.VMEM((B_PER_W,), jnp.int32),
            pltpu.VMEM((B_PER_W,), jnp.int32),
            pltpu.VMEM((B_PER_W,), jnp.int32),
            pltpu.SemaphoreType.DMA,
        ],
        compiler_params=pltpu.CompilerParams(
            needs_layout_passes=False, use_tc_tiling_on_sc=False,
            disable_bounds_checks=True, disable_semaphore_checks=True,
            skip_device_barrier=True),
    )
    tab = jnp.concatenate(
        [hour_table.reshape(-1), day_table.reshape(-1),
         month_table.reshape(-1)])
    return run(x.reshape(-1), tab)


_kernel_impl.__name__ = "kernel"
_kernel_impl.__qualname__ = "kernel"


@functools.cache
def _jitted():
    fmt = jex_layout.Format(
        jex_layout.Layout(major_to_minor=(0, 1), tiling=()),
        jax.sharding.SingleDeviceSharding(jax.devices()[0]))
    return jax.jit(_kernel_impl, out_shardings=fmt)


def kernel(x, hour_table, day_table, month_table):
    return _jitted()(x, hour_table, day_table, month_table)
